# baseline (speedup 1.0000x reference)
import functools

import jax
import jax.numpy as jnp
from jax import lax
from jax.experimental import pallas as pl
from jax.experimental.pallas import tpu as pltpu
from jax.experimental.pallas import tpu_sc as plsc

EPS = 1e-5

_NC = 2
_NS = 16
_NW = _NC * _NS


def _make_gather(B, D1, D2, D3):
    bpw = B // _NW
    mesh = plsc.VectorSubcoreMesh(core_axis_name="c", subcore_axis_name="s")

    @functools.partial(
        pl.kernel,
        mesh=mesh,
        out_type=(
            jax.ShapeDtypeStruct((B, D1), jnp.float32),
            jax.ShapeDtypeStruct((B, D2), jnp.float32),
            jax.ShapeDtypeStruct((B, D3), jnp.float32),
        ),
        scratch_types=[
            pltpu.VMEM((bpw,), jnp.int32),
            pltpu.VMEM((bpw,), jnp.int32),
            pltpu.VMEM((bpw,), jnp.int32),
            pltpu.VMEM((bpw, D1), jnp.float32),
            pltpu.VMEM((bpw, D2), jnp.float32),
            pltpu.VMEM((bpw, D3), jnp.float32),
            pltpu.SemaphoreType.DMA,
        ],
        compiler_params=pltpu.CompilerParams(use_tc_tiling_on_sc=False),
    )
    def gather_k(i0_hbm, i1_hbm, i2_hbm, t1_hbm, t2_hbm, t3_hbm,
                 o1_hbm, o2_hbm, o3_hbm,
                 i0_v, i1_v, i2_v, r1_v, r2_v, r3_v, sem):
        wid = lax.axis_index("s") * _NC + lax.axis_index("c")
        base = wid * bpw
        pltpu.sync_copy(i0_hbm.at[pl.ds(base, bpw)], i0_v)
        pltpu.sync_copy(i1_hbm.at[pl.ds(base, bpw)], i1_v)
        pltpu.sync_copy(i2_hbm.at[pl.ds(base, bpw)], i2_v)
        c1 = pltpu.async_copy(t1_hbm.at[i0_v], r1_v, sem)
        c2 = pltpu.async_copy(t2_hbm.at[i1_v], r2_v, sem)
        c3 = pltpu.async_copy(t3_hbm.at[i2_v], r3_v, sem)
        c1.wait()
        c2.wait()
        c3.wait()
        pltpu.sync_copy(r1_v, o1_hbm.at[pl.ds(base, bpw)])
        pltpu.sync_copy(r2_v, o2_hbm.at[pl.ds(base, bpw)])
        pltpu.sync_copy(r3_v, o3_hbm.at[pl.ds(base, bpw)])

    return gather_k


def _mlp_body(t1, t2, t3, g0, b0, w1t, b1, g1, be1, w2t, b2, g2, be2, wot, bo,
              out):
    x = jnp.concatenate([t1[...], t2[...], t3[...]], axis=1)
    m = jnp.mean(x, axis=0, keepdims=True)
    v = jnp.mean((x - m) * (x - m), axis=0, keepdims=True)
    x = (x - m) * lax.rsqrt(v + EPS) * g0[...] + b0[...]

    h = jnp.dot(x, w1t[...], preferred_element_type=jnp.float32) + b1[...]
    h = jnp.maximum(h, 0.0)
    m = jnp.mean(h, axis=0, keepdims=True)
    v = jnp.mean((h - m) * (h - m), axis=0, keepdims=True)
    h = (h - m) * lax.rsqrt(v + EPS) * g1[...] + be1[...]

    h = jnp.dot(h, w2t[...], preferred_element_type=jnp.float32) + b2[...]
    h = jnp.maximum(h, 0.0)
    m = jnp.mean(h, axis=0, keepdims=True)
    v = jnp.mean((h - m) * (h - m), axis=0, keepdims=True)
    h = (h - m) * lax.rsqrt(v + EPS) * g2[...] + be2[...]

    out[...] = jnp.dot(h, wot[...], preferred_element_type=jnp.float32) + bo[...]


def kernel(input, item_emb, cat_emb, shop_emb, g0, b0, W1, b1, g1, be1,
           W2, b2, g2, be2, Wo, bo):
    B = input.shape[0]
    idx = input.astype(jnp.int32)
    i0 = idx[:, 0]
    i1 = idx[:, 1]
    i2 = idx[:, 2]

    gather_k = _make_gather(B, item_emb.shape[1], cat_emb.shape[1],
                            shop_emb.shape[1])
    t1, t2, t3 = gather_k(i0, i1, i2, item_emb, cat_emb, shop_emb)

    mlp = pl.pallas_call(
        _mlp_body,
        out_shape=jax.ShapeDtypeStruct((B, 1), jnp.float32),
    )
    out = mlp(
        t1, t2, t3,
        g0.reshape(1, -1), b0.reshape(1, -1),
        W1.T, b1.reshape(1, -1), g1.reshape(1, -1), be1.reshape(1, -1),
        W2.T, b2.reshape(1, -1), g2.reshape(1, -1), be2.reshape(1, -1),
        Wo.T, bo.reshape(1, -1),
    )
    return out[:, 0]


# speedup vs baseline: 2.0744x; 2.0744x over previous
import functools

import jax
import jax.numpy as jnp
from jax import lax
from jax.experimental import pallas as pl
from jax.experimental.pallas import tpu as pltpu
from jax.experimental.pallas import tpu_sc as plsc

EPS = 1e-5

_NC = 2
_NS = 16
_NW = _NC * _NS


def _make_gather(B, V, D1, D2, D3):
    bpw = B // _NW
    D = D1 + D2 + D3
    feat = [(0, c * V) for c in range(D1)]
    feat += [(1, (D1 + c) * V) for c in range(D2)]
    feat += [(2, (D1 + D2 + c) * V) for c in range(D3)]

    mesh = plsc.VectorSubcoreMesh(core_axis_name="c", subcore_axis_name="s")

    @functools.partial(
        pl.kernel,
        mesh=mesh,
        out_type=jax.ShapeDtypeStruct((D, B), jnp.float32),
        scratch_types=[
            pltpu.VMEM((bpw, 3), jnp.int32),
            pltpu.VMEM((D, bpw), jnp.int32),
            pltpu.VMEM((D, bpw), jnp.float32),
            pltpu.SemaphoreType.DMA,
        ],
        compiler_params=pltpu.CompilerParams(
            use_tc_tiling_on_sc=False, needs_layout_passes=False
        ),
    )
    def gather_k(inp_hbm, tall_hbm, out_hbm, idx3_v, idxg_v, xt_v, sem):
        wid = lax.axis_index("s") * _NC + lax.axis_index("c")
        base = wid * bpw
        pltpu.sync_copy(inp_hbm.at[pl.ds(base, bpw)], idx3_v)

        for c, (src_col, off) in enumerate(feat):
            col_v = jnp.full((16,), src_col, jnp.int32)

            def body(k, _, c=c, col_v=col_v, off=off):
                rows = lax.iota(jnp.int32, 16) + 16 * k
                vals = plsc.load_gather(idx3_v, [rows, col_v])
                idxg_v[c, pl.ds(16 * k, 16)] = vals + off
                return 0

            lax.fori_loop(0, bpw // 16, body, 0)

        copies = [
            pltpu.async_copy(tall_hbm.at[idxg_v.at[c]], xt_v.at[c], sem)
            for c in range(D)
        ]
        for cp in copies:
            cp.wait()
        pltpu.sync_copy(xt_v, out_hbm.at[:, pl.ds(base, bpw)])

    return gather_k


def _mlp_body(xt, g0, b0, w1, b1, g1, be1, w2, b2, g2, be2, wo, bo, out):
    x = xt[...]
    m = jnp.mean(x, axis=1, keepdims=True)
    v = jnp.mean((x - m) * (x - m), axis=1, keepdims=True)
    x = (x - m) * lax.rsqrt(v + EPS) * g0[...] + b0[...]

    h = jnp.dot(w1[...], x, preferred_element_type=jnp.float32) + b1[...]
    h = jnp.maximum(h, 0.0)
    m = jnp.mean(h, axis=1, keepdims=True)
    v = jnp.mean((h - m) * (h - m), axis=1, keepdims=True)
    h = (h - m) * lax.rsqrt(v + EPS) * g1[...] + be1[...]

    h = jnp.dot(w2[...], h, preferred_element_type=jnp.float32) + b2[...]
    h = jnp.maximum(h, 0.0)
    m = jnp.mean(h, axis=1, keepdims=True)
    v = jnp.mean((h - m) * (h - m), axis=1, keepdims=True)
    h = (h - m) * lax.rsqrt(v + EPS) * g2[...] + be2[...]

    out[...] = jnp.sum(h * wo[...], axis=0) + bo[...]


def kernel(input, item_emb, cat_emb, shop_emb, g0, b0, W1, b1, g1, be1,
           W2, b2, g2, be2, Wo, bo):
    B = input.shape[0]
    V, D1 = item_emb.shape
    D2 = cat_emb.shape[1]
    D3 = shop_emb.shape[1]

    idx = input.astype(jnp.int32)
    tall = jnp.concatenate(
        [item_emb.T.reshape(-1), cat_emb.T.reshape(-1), shop_emb.T.reshape(-1)]
    )

    gather_k = _make_gather(B, V, D1, D2, D3)
    xt = gather_k(idx, tall)

    mlp = pl.pallas_call(
        _mlp_body,
        out_shape=jax.ShapeDtypeStruct((B,), jnp.float32),
    )
    return mlp(
        xt,
        g0.reshape(-1, 1), b0.reshape(-1, 1),
        W1, b1.reshape(-1, 1), g1.reshape(-1, 1), be1.reshape(-1, 1),
        W2, b2.reshape(-1, 1), g2.reshape(-1, 1), be2.reshape(-1, 1),
        Wo.T, bo,
    )


# speedup vs baseline: 2.6371x; 1.2712x over previous
import functools

import jax
import jax.numpy as jnp
from jax import lax
from jax.experimental import pallas as pl
from jax.experimental.pallas import tpu as pltpu
from jax.experimental.pallas import tpu_sc as plsc

EPS = 1e-5

_NC = 2
_NS = 16
_NW = _NC * _NS


def _make_gather(B, V, D1, D2, D3):
    bpw = B // _NW
    D = D1 + D2 + D3
    feat = [(0, 0, c * V) for c in range(D1)]
    feat += [(1, 1, c * V) for c in range(D2)]
    feat += [(2, 2, c * V) for c in range(D3)]

    mesh = plsc.VectorSubcoreMesh(core_axis_name="c", subcore_axis_name="s")

    @functools.partial(
        pl.kernel,
        mesh=mesh,
        out_type=jax.ShapeDtypeStruct((D, B), jnp.float32),
        scratch_types=[
            pltpu.VMEM((bpw, 3), jnp.int32),
            pltpu.VMEM((D, bpw), jnp.int32),
            pltpu.VMEM((D, bpw), jnp.float32),
            pltpu.SemaphoreType.DMA,
        ],
        compiler_params=pltpu.CompilerParams(
            use_tc_tiling_on_sc=False, needs_layout_passes=False
        ),
    )
    def gather_k(inp_hbm, t1_hbm, t2_hbm, t3_hbm, out_hbm,
                 idx3_v, idxg_v, xt_v, sem):
        tables = (t1_hbm, t2_hbm, t3_hbm)
        wid = lax.axis_index("s") * _NC + lax.axis_index("c")
        base = wid * bpw
        pltpu.sync_copy(inp_hbm.at[pl.ds(base, bpw)], idx3_v)

        copies = []
        for c, (src_col, tid, off) in enumerate(feat):
            col_v = jnp.full((16,), src_col, jnp.int32)

            def body(k, _, c=c, col_v=col_v, off=off):
                rows = lax.iota(jnp.int32, 16) + 16 * k
                vals = plsc.load_gather(idx3_v, [rows, col_v])
                idxg_v[c, pl.ds(16 * k, 16)] = vals + off
                return 0

            lax.fori_loop(0, bpw // 16, body, 0)
            copies.append(
                pltpu.async_copy(tables[tid].at[idxg_v.at[c]], xt_v.at[c], sem)
            )
        for cp in copies:
            cp.wait()
        pltpu.sync_copy(xt_v, out_hbm.at[:, pl.ds(base, bpw)])

    return gather_k


def _mlp_body(xt, g0, b0, w1, b1, g1, be1, w2, b2, g2, be2, wo, bo, out):
    def col(r):
        return r[...][:, None]

    x = xt[...]
    m = jnp.mean(x, axis=1, keepdims=True)
    v = jnp.mean((x - m) * (x - m), axis=1, keepdims=True)
    x = (x - m) * lax.rsqrt(v + EPS) * col(g0) + col(b0)

    h = jnp.dot(w1[...], x, preferred_element_type=jnp.float32) + col(b1)
    h = jnp.maximum(h, 0.0)
    m = jnp.mean(h, axis=1, keepdims=True)
    v = jnp.mean((h - m) * (h - m), axis=1, keepdims=True)
    h = (h - m) * lax.rsqrt(v + EPS) * col(g1) + col(be1)

    h = jnp.dot(w2[...], h, preferred_element_type=jnp.float32) + col(b2)
    h = jnp.maximum(h, 0.0)
    m = jnp.mean(h, axis=1, keepdims=True)
    v = jnp.mean((h - m) * (h - m), axis=1, keepdims=True)
    h = (h - m) * lax.rsqrt(v + EPS) * col(g2) + col(be2)

    out[...] = jnp.sum(h * wo[...].T, axis=0) + bo[...]


def kernel(input, item_emb, cat_emb, shop_emb, g0, b0, W1, b1, g1, be1,
           W2, b2, g2, be2, Wo, bo):
    B = input.shape[0]
    V, D1 = item_emb.shape
    D2 = cat_emb.shape[1]
    D3 = shop_emb.shape[1]

    idx = input.astype(jnp.int32)
    t1f = item_emb.T.reshape(-1)
    t2f = cat_emb.T.reshape(-1)
    t3f = shop_emb.T.reshape(-1)

    gather_k = _make_gather(B, V, D1, D2, D3)
    xt = gather_k(idx, t1f, t2f, t3f)

    mlp = pl.pallas_call(
        _mlp_body,
        out_shape=jax.ShapeDtypeStruct((B,), jnp.float32),
    )
    return mlp(xt, g0, b0, W1, b1, g1, be1, W2, b2, g2, be2, Wo, bo)


# speedup vs baseline: 2.6569x; 1.0075x over previous
import functools

import jax
import jax.numpy as jnp
from jax import lax
from jax.experimental import pallas as pl
from jax.experimental.pallas import tpu as pltpu
from jax.experimental.pallas import tpu_sc as plsc

EPS = 1e-5

_NC = 2
_NS = 16
_NW = _NC * _NS


def _make_gather(B, V, D1, D2, D3):
    bpw = B // _NW
    D = D1 + D2 + D3
    feat = [(0, 0, c) for c in range(D1)]
    feat += [(1, 1, c) for c in range(D2)]
    feat += [(2, 2, c) for c in range(D3)]

    mesh = plsc.VectorSubcoreMesh(core_axis_name="c", subcore_axis_name="s")

    @functools.partial(
        pl.kernel,
        mesh=mesh,
        out_type=jax.ShapeDtypeStruct((D, B), jnp.float32),
        scratch_types=[
            pltpu.VMEM((bpw, 3), jnp.int32),
            pltpu.VMEM((3, bpw), jnp.int32),
            pltpu.VMEM((D, bpw), jnp.float32),
            pltpu.SemaphoreType.DMA,
        ],
        compiler_params=pltpu.CompilerParams(
            use_tc_tiling_on_sc=False, needs_layout_passes=False
        ),
    )
    def gather_k(inp_hbm, t1_hbm, t2_hbm, t3_hbm, out_hbm,
                 idx3_v, idxb_v, xt_v, sem):
        tables = (t1_hbm, t2_hbm, t3_hbm)
        wid = lax.axis_index("s") * _NC + lax.axis_index("c")
        base = wid * bpw
        pltpu.sync_copy(inp_hbm.at[pl.ds(base, bpw)], idx3_v)

        for s in range(3):
            col_v = jnp.full((16,), s, jnp.int32)

            def body(k, _, s=s, col_v=col_v):
                rows = lax.iota(jnp.int32, 16) + 16 * k
                idxb_v[s, pl.ds(16 * k, 16)] = plsc.load_gather(
                    idx3_v, [rows, col_v])
                return 0

            lax.fori_loop(0, bpw // 16, body, 0)

        copies = [
            pltpu.async_copy(
                tables[tid].at[c_local].at[idxb_v.at[src_col]],
                xt_v.at[c], sem)
            for c, (src_col, tid, c_local) in enumerate(feat)
        ]
        for cp in copies:
            cp.wait()
        pltpu.sync_copy(xt_v, out_hbm.at[:, pl.ds(base, bpw)])

    return gather_k


def _mlp_body(xt, g0, b0, w1, b1, g1, be1, w2, b2, g2, be2, wo, bo, out):
    def col(r):
        return r[...][:, None]

    x = xt[...]
    m = jnp.mean(x, axis=1, keepdims=True)
    v = jnp.mean((x - m) * (x - m), axis=1, keepdims=True)
    x = (x - m) * lax.rsqrt(v + EPS) * col(g0) + col(b0)

    h = jnp.dot(w1[...], x, preferred_element_type=jnp.float32) + col(b1)
    h = jnp.maximum(h, 0.0)
    m = jnp.mean(h, axis=1, keepdims=True)
    v = jnp.mean((h - m) * (h - m), axis=1, keepdims=True)
    h = (h - m) * lax.rsqrt(v + EPS) * col(g1) + col(be1)

    h = jnp.dot(w2[...], h, preferred_element_type=jnp.float32) + col(b2)
    h = jnp.maximum(h, 0.0)
    m = jnp.mean(h, axis=1, keepdims=True)
    v = jnp.mean((h - m) * (h - m), axis=1, keepdims=True)
    h = (h - m) * lax.rsqrt(v + EPS) * col(g2) + col(be2)

    out[...] = jnp.sum(h * wo[...].T, axis=0) + bo[...]


def kernel(input, item_emb, cat_emb, shop_emb, g0, b0, W1, b1, g1, be1,
           W2, b2, g2, be2, Wo, bo):
    B = input.shape[0]
    V, D1 = item_emb.shape
    D2 = cat_emb.shape[1]
    D3 = shop_emb.shape[1]

    idx = input.astype(jnp.int32)

    gather_k = _make_gather(B, V, D1, D2, D3)
    xt = gather_k(idx, item_emb.T, cat_emb.T, shop_emb.T)

    mlp = pl.pallas_call(
        _mlp_body,
        out_shape=jax.ShapeDtypeStruct((B,), jnp.float32),
    )
    return mlp(xt, g0, b0, W1, b1, g1, be1, W2, b2, g2, be2, Wo, bo)


# speedup vs baseline: 3.2830x; 1.2356x over previous
import functools

import jax
import jax.numpy as jnp
from jax import lax
from jax.experimental import pallas as pl
from jax.experimental.pallas import tpu as pltpu
from jax.experimental.pallas import tpu_sc as plsc

EPS = 1e-5

_NC = 2
_NS = 16
_NW = _NC * _NS


def _make_gather(B, V, D1, D2, D3):
    bpw = B // _NW
    D = D1 + D2 + D3
    feat = [(0, c) for c in range(D1)]
    feat += [(1, c) for c in range(D2)]
    feat += [(2, c) for c in range(D3)]

    mesh = plsc.VectorSubcoreMesh(core_axis_name="c", subcore_axis_name="s")

    @functools.partial(
        pl.kernel,
        mesh=mesh,
        out_type=jax.ShapeDtypeStruct((D, B), jnp.float32),
        scratch_types=[
            pltpu.VMEM((3, bpw), jnp.int32),
            pltpu.VMEM((D, bpw), jnp.float32),
            pltpu.SemaphoreType.DMA,
        ],
        compiler_params=pltpu.CompilerParams(
            use_tc_tiling_on_sc=False, needs_layout_passes=False
        ),
    )
    def gather_k(i0_hbm, i1_hbm, i2_hbm, t1_hbm, t2_hbm, t3_hbm, out_hbm,
                 idxb_v, xt_v, sem):
        tables = (t1_hbm, t2_hbm, t3_hbm)
        wid = lax.axis_index("s") * _NC + lax.axis_index("c")
        base = wid * bpw
        for s, i_hbm in enumerate((i0_hbm, i1_hbm, i2_hbm)):
            pltpu.sync_copy(i_hbm.at[pl.ds(base, bpw)], idxb_v.at[s])

        copies = [
            pltpu.async_copy(
                tables[tid].at[c_local].at[idxb_v.at[tid]], xt_v.at[c], sem)
            for c, (tid, c_local) in enumerate(feat)
        ]
        for cp in copies:
            cp.wait()
        pltpu.sync_copy(xt_v, out_hbm.at[:, pl.ds(base, bpw)])

    return gather_k


def _mlp_body(xt, g0, b0, w1, b1, g1, be1, w2, b2, g2, be2, wo, bo, out):
    n = out.shape[0] * out.shape[1]

    def bn(h, g, b):
        s1 = jnp.sum(h, axis=2)
        m = (jnp.sum(s1, axis=1) / n)[:, None, None]
        d = h - m
        s2 = jnp.sum(d * d, axis=2)
        v = (jnp.sum(s2, axis=1) / n)[:, None, None]
        return d * lax.rsqrt(v + EPS) * g[...][:, None, None] \
            + b[...][:, None, None]

    def mm(w, h):
        return lax.dot_general(
            w[...], h, dimension_numbers=(((1,), (0,)), ((), ())),
            preferred_element_type=jnp.float32)

    x = bn(xt[...], g0, b0)
    h = jnp.maximum(mm(w1, x) + b1[...][:, None, None], 0.0)
    h = bn(h, g1, be1)
    h = jnp.maximum(mm(w2, h) + b2[...][:, None, None], 0.0)
    h = bn(h, g2, be2)
    h = h * wo[...].T[:, :, None]
    out[...] = jnp.sum(h, axis=0) + bo[...][:, None]


def kernel(input, item_emb, cat_emb, shop_emb, g0, b0, W1, b1, g1, be1,
           W2, b2, g2, be2, Wo, bo):
    B = input.shape[0]
    V, D1 = item_emb.shape
    D2 = cat_emb.shape[1]
    D3 = shop_emb.shape[1]
    D = D1 + D2 + D3

    idx = input.astype(jnp.int32)
    i0 = idx[:, 0]
    i1 = idx[:, 1]
    i2 = idx[:, 2]

    gather_k = _make_gather(B, V, D1, D2, D3)
    xt = gather_k(i0, i1, i2, item_emb.T, cat_emb.T, shop_emb.T)
    xt3 = xt.reshape(D, 128, B // 128)

    mlp = pl.pallas_call(
        _mlp_body,
        out_shape=jax.ShapeDtypeStruct((128, B // 128), jnp.float32),
    )
    out = mlp(xt3, g0, b0, W1, b1, g1, be1, W2, b2, g2, be2, Wo, bo)
    return out.reshape(B)


# speedup vs baseline: 3.3588x; 1.0231x over previous
import functools

import jax
import jax.numpy as jnp
from jax import lax
from jax.experimental import pallas as pl
from jax.experimental.pallas import tpu as pltpu
from jax.experimental.pallas import tpu_sc as plsc

EPS = 1e-5

_NC = 2
_NS = 16
_NW = _NC * _NS


def _make_gather(B, dims):
    bpw = B // _NW
    T = len(dims)
    D = sum(dims)
    feat = [(t, c) for t, d in enumerate(dims) for c in range(d)]

    mesh = plsc.VectorSubcoreMesh(core_axis_name="c", subcore_axis_name="s")

    @functools.partial(
        pl.kernel,
        mesh=mesh,
        out_type=jax.ShapeDtypeStruct((D, B), jnp.float32),
        scratch_types=[
            pltpu.VMEM((T, bpw), jnp.int32),
            pltpu.VMEM((D, bpw), jnp.float32),
            pltpu.SemaphoreType.DMA,
        ],
        compiler_params=pltpu.CompilerParams(
            use_tc_tiling_on_sc=False, needs_layout_passes=False
        ),
    )
    def gather_k(*refs):
        i_hbms = refs[:T]
        tables = refs[T:2 * T]
        out_hbm = refs[2 * T]
        idxb_v, xt_v, sem = refs[2 * T + 1:]
        wid = lax.axis_index("s") * _NC + lax.axis_index("c")
        base = wid * bpw
        for s, i_hbm in enumerate(i_hbms):
            pltpu.sync_copy(i_hbm.at[pl.ds(base, bpw)], idxb_v.at[s])

        copies = [
            pltpu.async_copy(
                tables[tid].at[c_local].at[idxb_v.at[tid]], xt_v.at[c], sem)
            for c, (tid, c_local) in enumerate(feat)
        ]
        for cp in copies:
            cp.wait()
        pltpu.sync_copy(xt_v, out_hbm.at[:, pl.ds(base, bpw)])

    return gather_k


def _mlp_body(xt, xt2, g0, b0, w1, b1, g1, be1, w2, b2, g2, be2, wo, bo, out):
    n = out.shape[0] * out.shape[1]

    def bn(h, g, b):
        s1 = jnp.sum(h, axis=2)
        m = (jnp.sum(s1, axis=1) / n)[:, None, None]
        d = h - m
        s2 = jnp.sum(d * d, axis=2)
        v = (jnp.sum(s2, axis=1) / n)[:, None, None]
        return d * lax.rsqrt(v + EPS) * g[...][:, None, None] \
            + b[...][:, None, None]

    def mm(w, h):
        return lax.dot_general(
            w[...], h, dimension_numbers=(((1,), (0,)), ((), ())),
            preferred_element_type=jnp.float32)

    x = bn(jnp.concatenate([xt[...], xt2[...]], axis=0), g0, b0)
    h = jnp.maximum(mm(w1, x) + b1[...][:, None, None], 0.0)
    h = bn(h, g1, be1)
    h = jnp.maximum(mm(w2, h) + b2[...][:, None, None], 0.0)
    h = bn(h, g2, be2)
    h = h * wo[...].T[:, :, None]
    out[...] = jnp.sum(h, axis=0) + bo[...][:, None]


def kernel(input, item_emb, cat_emb, shop_emb, g0, b0, W1, b1, g1, be1,
           W2, b2, g2, be2, Wo, bo):
    B = input.shape[0]
    V, D1 = item_emb.shape
    D2 = cat_emb.shape[1]
    D3 = shop_emb.shape[1]
    D = D1 + D2 + D3

    idx = input.astype(jnp.int32)
    i0 = idx[:, 0]
    i1 = idx[:, 1]
    i2 = idx[:, 2]

    gather_cs = _make_gather(B, (D2, D3))
    xt_cs = gather_cs(i1, i2, cat_emb.T, shop_emb.T)
    gather_it = _make_gather(B, (D1,))
    xt_it = gather_it(i0, item_emb.T)

    xt_it3 = xt_it.reshape(D1, 128, B // 128)
    xt_cs3 = xt_cs.reshape(D2 + D3, 128, B // 128)

    mlp = pl.pallas_call(
        _mlp_body,
        out_shape=jax.ShapeDtypeStruct((128, B // 128), jnp.float32),
    )
    out = mlp(xt_it3, xt_cs3, g0, b0, W1, b1, g1, be1, W2, b2, g2, be2,
              Wo, bo)
    return out.reshape(B)


# speedup vs baseline: 3.3773x; 1.0055x over previous
import functools

import jax
import jax.numpy as jnp
from jax import lax
from jax.experimental import pallas as pl
from jax.experimental.pallas import tpu as pltpu
from jax.experimental.pallas import tpu_sc as plsc

EPS = 1e-5

_NC = 2
_NS = 16
_NW = _NC * _NS


def _make_gather(B, dims):
    bpw = B // _NW
    T = len(dims)
    D = sum(dims)
    feat = [(t, c) for t, d in enumerate(dims) for c in range(d)]

    mesh = plsc.VectorSubcoreMesh(core_axis_name="c", subcore_axis_name="s")

    @functools.partial(
        pl.kernel,
        mesh=mesh,
        out_type=jax.ShapeDtypeStruct((D, B), jnp.float32),
        scratch_types=[
            pltpu.VMEM((T, bpw), jnp.int32),
            pltpu.VMEM((D, bpw), jnp.float32),
            pltpu.SemaphoreType.DMA,
        ],
        compiler_params=pltpu.CompilerParams(
            use_tc_tiling_on_sc=False, needs_layout_passes=False
        ),
    )
    def gather_k(*refs):
        i_hbms = refs[:T]
        tables = refs[T:2 * T]
        out_hbm = refs[2 * T]
        idxb_v, xt_v, sem = refs[2 * T + 1:]
        wid = lax.axis_index("s") * _NC + lax.axis_index("c")
        base = wid * bpw
        for s, i_hbm in enumerate(i_hbms):
            pltpu.sync_copy(i_hbm.at[pl.ds(base, bpw)], idxb_v.at[s])

        copies = [
            pltpu.async_copy(
                tables[tid].at[c_local].at[idxb_v.at[tid]], xt_v.at[c], sem)
            for c, (tid, c_local) in enumerate(feat)
        ]
        for cp in copies:
            cp.wait()
        pltpu.sync_copy(xt_v, out_hbm.at[:, pl.ds(base, bpw)])

    return gather_k


def _mlp_body(xt, xt2, g0, b0, w1, b1, g1, be1, w2, b2, g2, be2, wo, bo, out):
    n = out.shape[0] * out.shape[1]

    def stats(h):
        m = (jnp.sum(jnp.sum(h, axis=2), axis=1) / n)
        m2 = (jnp.sum(jnp.sum(h * h, axis=2), axis=1) / n)
        s = lax.rsqrt(jnp.maximum(m2 - m * m, 0.0) + EPS)
        return m, s

    def mm(w, h):
        return lax.dot_general(
            w, h, dimension_numbers=(((1,), (0,)), ((), ())),
            preferred_element_type=jnp.float32)

    x = jnp.concatenate([xt[...], xt2[...]], axis=0)
    m, s = stats(x)
    a = g0[...] * s
    w1e = w1[...] * a[None, :]
    b1e = mm(w1[...], (b0[...] - m * a)[:, None]) + b1[...][:, None]
    h = jnp.maximum(mm(w1e, x) + b1e[:, :, None], 0.0)

    m, s = stats(h)
    a = g1[...] * s
    w2e = w2[...] * a[None, :]
    b2e = mm(w2[...], (be1[...] - m * a)[:, None]) + b2[...][:, None]
    h = jnp.maximum(mm(w2e, h) + b2e[:, :, None], 0.0)

    m, s = stats(h)
    a = g2[...] * s
    woe = (wo[...][0] * a)[:, None, None]
    boe = jnp.sum(wo[...][0] * (be2[...] - m * a)) + bo[...]
    out[...] = jnp.sum(h * woe, axis=0) + boe[:, None]


def kernel(input, item_emb, cat_emb, shop_emb, g0, b0, W1, b1, g1, be1,
           W2, b2, g2, be2, Wo, bo):
    B = input.shape[0]
    V, D1 = item_emb.shape
    D2 = cat_emb.shape[1]
    D3 = shop_emb.shape[1]
    D = D1 + D2 + D3

    idx = input.astype(jnp.int32)
    i0 = idx[:, 0]
    i1 = idx[:, 1]
    i2 = idx[:, 2]

    gather_cs = _make_gather(B, (D2, D3))
    xt_cs = gather_cs(i1, i2, cat_emb.T, shop_emb.T)
    gather_it = _make_gather(B, (D1,))
    xt_it = gather_it(i0, item_emb.T)

    xt_it3 = xt_it.reshape(D1, 128, B // 128)
    xt_cs3 = xt_cs.reshape(D2 + D3, 128, B // 128)

    mlp = pl.pallas_call(
        _mlp_body,
        out_shape=jax.ShapeDtypeStruct((128, B // 128), jnp.float32),
    )
    out = mlp(xt_it3, xt_cs3, g0, b0, W1, b1, g1, be1, W2, b2, g2, be2,
              Wo, bo)
    return out.reshape(B)
